# baseline (speedup 1.0000x reference)
import functools

import jax
import jax.numpy as jnp
from jax import lax
from jax.experimental import pallas as pl
from jax.experimental.pallas import tpu as pltpu
from jax.experimental.pallas import tpu_sc as plsc

LATENT = 256
KCODES = 1024
ROWS = 8192
BLOCK = 1024
GRID = ROWS // BLOCK

NC = 2
NS = 16
NW = NC * NS
BPW = ROWS // NW


def _xla_row_sum(xx):
    s1 = xx[:, :128] + xx[:, 128:]
    acc = s1[:, 0:8] + s1[:, 8:16]
    for k in range(2, 16):
        acc = acc + s1[:, 8 * k:8 * k + 8]
    b = acc[:, :4] + acc[:, 4:]
    c = b[:, :2] + b[:, 2:]
    return c[:, 0:1] + c[:, 1:2]


def _first_argmin(dists):
    m = jnp.min(dists, axis=1, keepdims=True)
    iota = lax.broadcasted_iota(jnp.int32, dists.shape, 1)
    return jnp.min(jnp.where(dists == m, iota, KCODES), axis=1)


def _tc_body(x_ref, emb_ref, w_ref, ids_ref, loss_ref, cb_ref, c2_ref):
    i = pl.program_id(0)

    @pl.when(i == 0)
    def _():
        cb = lax.dot_general(
            emb_ref[...], w_ref[...],
            (((1,), (1,)), ((), ())),
            preferred_element_type=jnp.float32)
        cb_ref[...] = cb
        ones = jnp.ones((1, LATENT), jnp.float32)
        c2_ref[...] = lax.dot_general(
            ones, cb * cb,
            (((1,), (1,)), ((), ())),
            preferred_element_type=jnp.float32)
        loss_ref[0, 0] = 0.0

    xb = x_ref[...]
    x2 = _xla_row_sum(xb * xb)
    prod = lax.dot_general(
        xb, cb_ref[...],
        (((1,), (1,)), ((), ())),
        preferred_element_type=jnp.float32)
    d2 = jnp.maximum(x2 + c2_ref[...] - 2.0 * prod, 0.0)
    dists = jnp.sqrt(d2)
    ids_ref[0, 0, :] = _first_argmin(dists)
    loss_ref[0, 0] += jnp.sum(jnp.min(d2, axis=1))


def _tc_call(x_flat, embedding, W_proj):
    return pl.pallas_call(
        _tc_body,
        grid=(GRID,),
        in_specs=[
            pl.BlockSpec((BLOCK, LATENT), lambda i: (i, 0)),
            pl.BlockSpec((KCODES, LATENT), lambda i: (0, 0)),
            pl.BlockSpec((LATENT, LATENT), lambda i: (0, 0)),
        ],
        out_specs=[
            pl.BlockSpec((1, 1, BLOCK), lambda i: (i, 0, 0)),
            pl.BlockSpec((1, 1), lambda i: (0, 0),
                         memory_space=pltpu.SMEM),
            pl.BlockSpec((KCODES, LATENT), lambda i: (0, 0)),
        ],
        out_shape=[
            jax.ShapeDtypeStruct((GRID, 1, BLOCK), jnp.int32),
            jax.ShapeDtypeStruct((1, 1), jnp.float32),
            jax.ShapeDtypeStruct((KCODES, LATENT), jnp.float32),
        ],
        scratch_shapes=[pltpu.VMEM((1, KCODES), jnp.float32)],
    )(x_flat, embedding, W_proj)


@functools.partial(
    pl.kernel,
    mesh=plsc.VectorSubcoreMesh(core_axis_name="c", subcore_axis_name="s"),
    out_type=jax.ShapeDtypeStruct((ROWS, LATENT), jnp.float32),
    scratch_types=[
        pltpu.VMEM((BPW,), jnp.int32),
        pltpu.VMEM((BPW, LATENT), jnp.float32),
        pltpu.SemaphoreType.DMA,
    ],
)
def _sc_gather(cb_hbm, idx_hbm, out_hbm, idx_v, rows_v, sem):
    wid = lax.axis_index("s") * NC + lax.axis_index("c")
    base = wid * BPW
    pltpu.sync_copy(idx_hbm.at[pl.ds(base, BPW)], idx_v)
    pltpu.async_copy(cb_hbm.at[idx_v], rows_v, sem).wait()
    pltpu.sync_copy(rows_v, out_hbm.at[pl.ds(base, BPW)])


def kernel(x, embedding, W_proj):
    x_flat = x.reshape(-1, LATENT)
    ids3, loss_acc, codebook = _tc_call(x_flat, embedding, W_proj)
    ids_flat = ids3.reshape(ROWS)
    quantized = _sc_gather(codebook, ids_flat)
    q_ste = quantized.reshape(x.shape)
    ids = ids3.reshape(x.shape[:-1])
    loss = 1.25 * loss_acc[0, 0] / (ROWS * LATENT)
    return q_ste, ids, loss


# speedup vs baseline: 1.0291x; 1.0291x over previous
import functools

import jax
import jax.numpy as jnp
from jax import lax
from jax.experimental import pallas as pl
from jax.experimental.pallas import tpu as pltpu
from jax.experimental.pallas import tpu_sc as plsc

LATENT = 256
KCODES = 1024
ROWS = 8192
BLOCK = 1024
GRID = ROWS // BLOCK

NC = 2
NS = 16
NW = NC * NS
BPW = ROWS // NW


def _xla_row_sum(xx):
    s1 = xx[:, :128] + xx[:, 128:]
    acc = s1[:, 0:8] + s1[:, 8:16]
    for k in range(2, 16):
        acc = acc + s1[:, 8 * k:8 * k + 8]
    b = acc[:, :4] + acc[:, 4:]
    c = b[:, :2] + b[:, 2:]
    return c[:, 0:1] + c[:, 1:2]


def _first_argmin(dists):
    m = jnp.min(dists, axis=1, keepdims=True)
    iota = lax.broadcasted_iota(jnp.int32, dists.shape, 1)
    return jnp.min(jnp.where(dists == m, iota, KCODES), axis=1), m


def _tc_body(x_ref, emb_ref, w_ref, ids_ref, loss_ref, cb_ref, c2_ref):
    i = pl.program_id(0)

    @pl.when(i == 0)
    def _():
        cb = lax.dot_general(
            emb_ref[...], w_ref[...],
            (((1,), (1,)), ((), ())),
            preferred_element_type=jnp.float32)
        cb_ref[...] = cb
        ones = jnp.ones((1, LATENT), jnp.float32)
        c2_ref[...] = lax.dot_general(
            ones, cb * cb,
            (((1,), (1,)), ((), ())),
            preferred_element_type=jnp.float32)
        loss_ref[0, 0] = 0.0

    xb = x_ref[...]
    x2 = _xla_row_sum(xb * xb)
    prod = lax.dot_general(
        xb, cb_ref[...],
        (((1,), (1,)), ((), ())),
        preferred_element_type=jnp.float32)
    d2 = jnp.maximum(x2 + c2_ref[...] - 2.0 * prod, 0.0)
    dists = jnp.sqrt(d2)
    ids, m = _first_argmin(dists)
    ids_ref[0, 0, :] = ids
    loss_ref[0, 0] += jnp.sum(m * m)


def _tc_call(x_flat, embedding, W_proj):
    return pl.pallas_call(
        _tc_body,
        grid=(GRID,),
        in_specs=[
            pl.BlockSpec((BLOCK, LATENT), lambda i: (i, 0)),
            pl.BlockSpec((KCODES, LATENT), lambda i: (0, 0)),
            pl.BlockSpec((LATENT, LATENT), lambda i: (0, 0)),
        ],
        out_specs=[
            pl.BlockSpec((1, 1, BLOCK), lambda i: (i, 0, 0)),
            pl.BlockSpec((1, 1), lambda i: (0, 0),
                         memory_space=pltpu.SMEM),
            pl.BlockSpec((KCODES, LATENT), lambda i: (0, 0)),
        ],
        out_shape=[
            jax.ShapeDtypeStruct((GRID, 1, BLOCK), jnp.int32),
            jax.ShapeDtypeStruct((1, 1), jnp.float32),
            jax.ShapeDtypeStruct((KCODES, LATENT), jnp.float32),
        ],
        scratch_shapes=[pltpu.VMEM((1, KCODES), jnp.float32)],
    )(x_flat, embedding, W_proj)


@functools.partial(
    pl.kernel,
    mesh=plsc.VectorSubcoreMesh(core_axis_name="c", subcore_axis_name="s"),
    out_type=jax.ShapeDtypeStruct((ROWS, LATENT), jnp.float32),
    scratch_types=[
        pltpu.VMEM((BPW,), jnp.int32),
        pltpu.VMEM((BPW, LATENT), jnp.float32),
        pltpu.SemaphoreType.DMA,
    ],
)
def _sc_gather(cb_hbm, idx_hbm, out_hbm, idx_v, rows_v, sem):
    wid = lax.axis_index("s") * NC + lax.axis_index("c")
    base = wid * BPW
    pltpu.sync_copy(idx_hbm.at[pl.ds(base, BPW)], idx_v)
    pltpu.async_copy(cb_hbm.at[idx_v], rows_v, sem).wait()
    pltpu.sync_copy(rows_v, out_hbm.at[pl.ds(base, BPW)])


def kernel(x, embedding, W_proj):
    x_flat = x.reshape(-1, LATENT)
    ids3, loss_acc, codebook = _tc_call(x_flat, embedding, W_proj)
    ids_flat = ids3.reshape(ROWS)
    quantized = _sc_gather(codebook, ids_flat)
    q_ste = quantized.reshape(x.shape)
    ids = ids3.reshape(x.shape[:-1])
    loss = 1.25 * loss_acc[0, 0] / (ROWS * LATENT)
    return q_ste, ids, loss


# speedup vs baseline: 1.0458x; 1.0163x over previous
import functools

import jax
import jax.numpy as jnp
from jax import lax
from jax.experimental import pallas as pl
from jax.experimental.pallas import tpu as pltpu
from jax.experimental.pallas import tpu_sc as plsc

LATENT = 256
KCODES = 1024
ROWS = 8192
BLOCK = 1024
GRID = ROWS // BLOCK

NC = 2
NS = 16
NW = NC * NS
BPW = ROWS // NW


def _xla_row_sum(xx):
    s1 = xx[:, :128] + xx[:, 128:]
    acc = s1[:, 0:8] + s1[:, 8:16]
    for k in range(2, 16):
        acc = acc + s1[:, 8 * k:8 * k + 8]
    b = acc[:, :4] + acc[:, 4:]
    c = b[:, :2] + b[:, 2:]
    return c[:, 0:1] + c[:, 1:2]


def _first_argmin(dists):
    m = jnp.min(dists, axis=1, keepdims=True)
    iota = lax.broadcasted_iota(jnp.int32, dists.shape, 1)
    return jnp.min(jnp.where(dists == m, iota, KCODES), axis=1), m


def _cb_body(emb_ref, w_ref, cb_ref, c2_ref):
    cb = lax.dot_general(
        emb_ref[...], w_ref[...],
        (((1,), (1,)), ((), ())),
        preferred_element_type=jnp.float32)
    cb_ref[...] = cb
    ones = jnp.ones((1, LATENT), jnp.float32)
    c2_ref[...] = lax.dot_general(
        ones, cb * cb,
        (((1,), (1,)), ((), ())),
        preferred_element_type=jnp.float32)


def _cb_call(embedding, W_proj):
    return pl.pallas_call(
        _cb_body,
        out_shape=[
            jax.ShapeDtypeStruct((KCODES, LATENT), jnp.float32),
            jax.ShapeDtypeStruct((1, KCODES), jnp.float32),
        ],
    )(embedding, W_proj)


def _tc_body(x_ref, cb_ref, c2_ref, ids_ref, loss_ref):
    xb = x_ref[...]
    x2 = _xla_row_sum(xb * xb)
    prod = lax.dot_general(
        xb, cb_ref[...],
        (((1,), (1,)), ((), ())),
        preferred_element_type=jnp.float32)
    d2 = jnp.maximum(x2 + c2_ref[...] - 2.0 * prod, 0.0)
    dists = jnp.sqrt(d2)
    ids, m = _first_argmin(dists)
    ids_ref[0, 0, :] = ids
    loss_ref[0, 0, 0] = jnp.sum(m * m)


def _tc_call(x_flat, codebook, c2):
    return pl.pallas_call(
        _tc_body,
        grid=(GRID,),
        in_specs=[
            pl.BlockSpec((BLOCK, LATENT), lambda i: (i, 0)),
            pl.BlockSpec((KCODES, LATENT), lambda i: (0, 0)),
            pl.BlockSpec((1, KCODES), lambda i: (0, 0)),
        ],
        out_specs=[
            pl.BlockSpec((1, 1, BLOCK), lambda i: (i, 0, 0)),
            pl.BlockSpec((1, 1, 1), lambda i: (i, 0, 0),
                         memory_space=pltpu.SMEM),
        ],
        out_shape=[
            jax.ShapeDtypeStruct((GRID, 1, BLOCK), jnp.int32),
            jax.ShapeDtypeStruct((GRID, 1, 1), jnp.float32),
        ],
        compiler_params=pltpu.CompilerParams(
            dimension_semantics=("parallel",)),
    )(x_flat, codebook, c2)


@functools.partial(
    pl.kernel,
    mesh=plsc.VectorSubcoreMesh(core_axis_name="c", subcore_axis_name="s"),
    out_type=jax.ShapeDtypeStruct((ROWS, LATENT), jnp.float32),
    scratch_types=[
        pltpu.VMEM((BPW,), jnp.int32),
        pltpu.VMEM((BPW, LATENT), jnp.float32),
        pltpu.SemaphoreType.DMA,
    ],
)
def _sc_gather(cb_hbm, idx_hbm, out_hbm, idx_v, rows_v, sem):
    wid = lax.axis_index("s") * NC + lax.axis_index("c")
    base = wid * BPW
    pltpu.sync_copy(idx_hbm.at[pl.ds(base, BPW)], idx_v)
    pltpu.async_copy(cb_hbm.at[idx_v], rows_v, sem).wait()
    pltpu.sync_copy(rows_v, out_hbm.at[pl.ds(base, BPW)])


def kernel(x, embedding, W_proj):
    x_flat = x.reshape(-1, LATENT)
    codebook, c2 = _cb_call(embedding, W_proj)
    ids3, loss_parts = _tc_call(x_flat, codebook, c2)
    ids_flat = ids3.reshape(ROWS)
    quantized = _sc_gather(codebook, ids_flat)
    q_ste = quantized.reshape(x.shape)
    ids = ids3.reshape(x.shape[:-1])
    loss = 1.25 * jnp.sum(loss_parts) / (ROWS * LATENT)
    return q_ste, ids, loss


# speedup vs baseline: 1.1770x; 1.1254x over previous
import functools

import jax
import jax.numpy as jnp
from jax import lax
from jax.experimental import pallas as pl
from jax.experimental.pallas import tpu as pltpu
from jax.experimental.pallas import tpu_sc as plsc

LATENT = 256
KCODES = 1024
ROWS = 8192
BLOCK = 1024
GRID = ROWS // BLOCK
SUB = 64
NSUB = BLOCK // SUB

NC = 2
NS = 16
NW = NC * NS
BPW = ROWS // NW


def _xla_row_sum(xx):
    s1 = xx[:, :128] + xx[:, 128:]
    s1t = s1.T
    acc = s1t[0:8, :] + s1t[8:16, :]
    for k in range(2, 16):
        acc = acc + s1t[8 * k:8 * k + 8, :]
    b = acc[0:4, :] + acc[4:8, :]
    c = b[0:2, :] + b[2:4, :]
    return (c[0:1, :] + c[1:2, :]).T


def _tc_body(x_ref, emb_ref, w_ref, ids_ref, loss_ref, cb_ref,
             c2_ref, prod_ref, x2_ref, msq_ref):
    i = pl.program_id(0)

    @pl.when(i == 0)
    def _():
        cb = lax.dot_general(
            emb_ref[...], w_ref[...],
            (((1,), (1,)), ((), ())),
            preferred_element_type=jnp.float32)
        cb_ref[...] = cb
        ones = jnp.ones((1, LATENT), jnp.float32)
        c2_ref[...] = lax.dot_general(
            ones, cb * cb,
            (((1,), (1,)), ((), ())),
            preferred_element_type=jnp.float32)
        loss_ref[0, 0] = 0.0

    xb = x_ref[...]
    x2_ref[...] = _xla_row_sum(xb * xb)
    prod_ref[...] = lax.dot_general(
        xb, cb_ref[...],
        (((1,), (1,)), ((), ())),
        preferred_element_type=jnp.float32)

    c2 = c2_ref[...]
    for j in range(NSUB):
        p = prod_ref[j * SUB:(j + 1) * SUB, :]
        x2s = x2_ref[j * SUB:(j + 1) * SUB, :]
        d2 = jnp.maximum(x2s + c2 - 2.0 * p, 0.0)
        dists = d2 * lax.rsqrt(d2)
        m = jnp.min(dists, axis=1, keepdims=True)
        iota = lax.broadcasted_iota(jnp.int32, dists.shape, 1)
        ids = jnp.min(jnp.where(dists == m, iota, KCODES), axis=1)
        ids_ref[0, 0, j * SUB:(j + 1) * SUB] = ids
        msq_ref[j * SUB:(j + 1) * SUB, :] = m * m
    loss_ref[0, 0] += jnp.sum(msq_ref[...])


def _tc_call(x_flat, embedding, W_proj):
    return pl.pallas_call(
        _tc_body,
        grid=(GRID,),
        in_specs=[
            pl.BlockSpec((BLOCK, LATENT), lambda i: (i, 0)),
            pl.BlockSpec((KCODES, LATENT), lambda i: (0, 0)),
            pl.BlockSpec((LATENT, LATENT), lambda i: (0, 0)),
        ],
        out_specs=[
            pl.BlockSpec((1, 1, BLOCK), lambda i: (i, 0, 0)),
            pl.BlockSpec((1, 1), lambda i: (0, 0),
                         memory_space=pltpu.SMEM),
            pl.BlockSpec((KCODES, LATENT), lambda i: (0, 0)),
        ],
        out_shape=[
            jax.ShapeDtypeStruct((GRID, 1, BLOCK), jnp.int32),
            jax.ShapeDtypeStruct((1, 1), jnp.float32),
            jax.ShapeDtypeStruct((KCODES, LATENT), jnp.float32),
        ],
        scratch_shapes=[
            pltpu.VMEM((1, KCODES), jnp.float32),
            pltpu.VMEM((BLOCK, KCODES), jnp.float32),
            pltpu.VMEM((BLOCK, 1), jnp.float32),
            pltpu.VMEM((BLOCK, 1), jnp.float32),
        ],
    )(x_flat, embedding, W_proj)


@functools.partial(
    pl.kernel,
    mesh=plsc.VectorSubcoreMesh(core_axis_name="c", subcore_axis_name="s"),
    out_type=jax.ShapeDtypeStruct((ROWS, LATENT), jnp.float32),
    scratch_types=[
        pltpu.VMEM((BPW,), jnp.int32),
        pltpu.VMEM((BPW, LATENT), jnp.float32),
        pltpu.SemaphoreType.DMA,
    ],
)
def _sc_gather(cb_hbm, idx_hbm, out_hbm, idx_v, rows_v, sem):
    wid = lax.axis_index("s") * NC + lax.axis_index("c")
    base = wid * BPW
    pltpu.sync_copy(idx_hbm.at[pl.ds(base, BPW)], idx_v)
    pltpu.async_copy(cb_hbm.at[idx_v], rows_v, sem).wait()
    pltpu.sync_copy(rows_v, out_hbm.at[pl.ds(base, BPW)])


def kernel(x, embedding, W_proj):
    x_flat = x.reshape(-1, LATENT)
    ids3, loss_acc, codebook = _tc_call(x_flat, embedding, W_proj)
    ids_flat = ids3.reshape(ROWS)
    quantized = _sc_gather(codebook, ids_flat)
    q_ste = quantized.reshape(x.shape)
    ids = ids3.reshape(x.shape[:-1])
    loss = 1.25 * loss_acc[0, 0] / (ROWS * LATENT)
    return q_ste, ids, loss


# speedup vs baseline: 1.2807x; 1.0882x over previous
import functools

import jax
import jax.numpy as jnp
from jax import lax
from jax.experimental import pallas as pl
from jax.experimental.pallas import tpu as pltpu
from jax.experimental.pallas import tpu_sc as plsc

LATENT = 256
KCODES = 1024
ROWS = 8192
BLOCK = 1024
GRID = ROWS // BLOCK
SUB = 64
NSUB = BLOCK // SUB

NC = 2
NS = 16
NW = NC * NS
BPW = ROWS // NW


def _xla_row_sum(xx):
    s1 = xx[:, :128] + xx[:, 128:]
    s1t = s1.T
    acc = s1t[0:8, :] + s1t[8:16, :]
    for k in range(2, 16):
        acc = acc + s1t[8 * k:8 * k + 8, :]
    b = acc[0:4, :] + acc[4:8, :]
    c = b[0:2, :] + b[2:4, :]
    return (c[0:1, :] + c[1:2, :]).T


def _tc_body(x_ref, emb_ref, w_ref, ids_ref, loss_ref, cb_ref,
             c2_ref, x2_ref, msq_ref):
    i = pl.program_id(0)

    @pl.when(i == 0)
    def _():
        cb = lax.dot_general(
            emb_ref[...], w_ref[...],
            (((1,), (1,)), ((), ())),
            preferred_element_type=jnp.float32)
        cb_ref[...] = cb
        ones = jnp.ones((1, LATENT), jnp.float32)
        c2_ref[...] = lax.dot_general(
            ones, cb * cb,
            (((1,), (1,)), ((), ())),
            preferred_element_type=jnp.float32)
        loss_ref[0, 0] = 0.0

    xb = x_ref[...]
    x2_ref[...] = _xla_row_sum(xb * xb)
    cb = cb_ref[...]

    c2 = c2_ref[...]
    for j in range(NSUB):
        p = lax.dot_general(
            xb[j * SUB:(j + 1) * SUB, :], cb,
            (((1,), (1,)), ((), ())),
            preferred_element_type=jnp.float32)
        x2s = x2_ref[j * SUB:(j + 1) * SUB, :]
        d2 = jnp.maximum(x2s + c2 - 2.0 * p, 0.0)
        dists = d2 * lax.rsqrt(d2)
        m = jnp.min(dists, axis=1, keepdims=True)
        iota = lax.broadcasted_iota(jnp.int32, dists.shape, 1)
        ids = jnp.min(jnp.where(dists == m, iota, KCODES), axis=1)
        ids_ref[0, 0, j * SUB:(j + 1) * SUB] = ids
        msq_ref[j * SUB:(j + 1) * SUB, :] = m * m
    loss_ref[0, 0] += jnp.sum(msq_ref[...])


def _tc_call(x_flat, embedding, W_proj):
    return pl.pallas_call(
        _tc_body,
        grid=(GRID,),
        in_specs=[
            pl.BlockSpec((BLOCK, LATENT), lambda i: (i, 0)),
            pl.BlockSpec((KCODES, LATENT), lambda i: (0, 0)),
            pl.BlockSpec((LATENT, LATENT), lambda i: (0, 0)),
        ],
        out_specs=[
            pl.BlockSpec((1, 1, BLOCK), lambda i: (i, 0, 0)),
            pl.BlockSpec((1, 1), lambda i: (0, 0),
                         memory_space=pltpu.SMEM),
            pl.BlockSpec((KCODES, LATENT), lambda i: (0, 0)),
        ],
        out_shape=[
            jax.ShapeDtypeStruct((GRID, 1, BLOCK), jnp.int32),
            jax.ShapeDtypeStruct((1, 1), jnp.float32),
            jax.ShapeDtypeStruct((KCODES, LATENT), jnp.float32),
        ],
        scratch_shapes=[
            pltpu.VMEM((1, KCODES), jnp.float32),
            pltpu.VMEM((BLOCK, 1), jnp.float32),
            pltpu.VMEM((BLOCK, 1), jnp.float32),
        ],
    )(x_flat, embedding, W_proj)


@functools.partial(
    pl.kernel,
    mesh=plsc.VectorSubcoreMesh(core_axis_name="c", subcore_axis_name="s"),
    out_type=jax.ShapeDtypeStruct((ROWS, LATENT), jnp.float32),
    scratch_types=[
        pltpu.VMEM((BPW,), jnp.int32),
        pltpu.VMEM((BPW, LATENT), jnp.float32),
        pltpu.SemaphoreType.DMA,
    ],
)
def _sc_gather(cb_hbm, idx_hbm, out_hbm, idx_v, rows_v, sem):
    wid = lax.axis_index("s") * NC + lax.axis_index("c")
    base = wid * BPW
    pltpu.sync_copy(idx_hbm.at[pl.ds(base, BPW)], idx_v)
    pltpu.async_copy(cb_hbm.at[idx_v], rows_v, sem).wait()
    pltpu.sync_copy(rows_v, out_hbm.at[pl.ds(base, BPW)])


def kernel(x, embedding, W_proj):
    x_flat = x.reshape(-1, LATENT)
    ids3, loss_acc, codebook = _tc_call(x_flat, embedding, W_proj)
    ids_flat = ids3.reshape(ROWS)
    quantized = _sc_gather(codebook, ids_flat)
    q_ste = quantized.reshape(x.shape)
    ids = ids3.reshape(x.shape[:-1])
    loss = 1.25 * loss_acc[0, 0] / (ROWS * LATENT)
    return q_ste, ids, loss


# speedup vs baseline: 1.3323x; 1.0403x over previous
import functools

import jax
import jax.numpy as jnp
from jax import lax
from jax.experimental import pallas as pl
from jax.experimental.pallas import tpu as pltpu
from jax.experimental.pallas import tpu_sc as plsc

LATENT = 256
KCODES = 1024
ROWS = 8192
BLOCK = 2048
GRID = ROWS // BLOCK
SUB = 64
NSUB = BLOCK // SUB

NC = 2
NS = 16
NW = NC * NS
BPW = ROWS // NW


def _xla_row_sum(xx):
    s1 = xx[:, :128] + xx[:, 128:]
    s1t = s1.T
    acc = s1t[0:8, :] + s1t[8:16, :]
    for k in range(2, 16):
        acc = acc + s1t[8 * k:8 * k + 8, :]
    b = acc[0:4, :] + acc[4:8, :]
    c = b[0:2, :] + b[2:4, :]
    return (c[0:1, :] + c[1:2, :]).T


def _tc_body(x_ref, emb_ref, w_ref, ids_ref, loss_ref, cb_ref,
             c2_ref, x2_ref, msq_ref):
    i = pl.program_id(0)

    @pl.when(i == 0)
    def _():
        cb = lax.dot_general(
            emb_ref[...], w_ref[...],
            (((1,), (1,)), ((), ())),
            preferred_element_type=jnp.float32)
        cb_ref[...] = cb
        ones = jnp.ones((1, LATENT), jnp.float32)
        c2_ref[...] = lax.dot_general(
            ones, cb * cb,
            (((1,), (1,)), ((), ())),
            preferred_element_type=jnp.float32)
        loss_ref[0, 0] = 0.0

    xb = x_ref[...]
    x2_ref[...] = _xla_row_sum(xb * xb)
    cb = cb_ref[...]

    c2 = c2_ref[...]
    for j in range(NSUB):
        p = lax.dot_general(
            xb[j * SUB:(j + 1) * SUB, :], cb,
            (((1,), (1,)), ((), ())),
            preferred_element_type=jnp.float32)
        x2s = x2_ref[j * SUB:(j + 1) * SUB, :]
        d2 = jnp.maximum(x2s + c2 - 2.0 * p, 0.0)
        dists = d2 * lax.rsqrt(d2)
        m = jnp.min(dists, axis=1, keepdims=True)
        iota = lax.broadcasted_iota(jnp.int32, dists.shape, 1)
        ids = jnp.min(jnp.where(dists == m, iota, KCODES), axis=1)
        ids_ref[0, 0, j * SUB:(j + 1) * SUB] = ids
        msq_ref[j * SUB:(j + 1) * SUB, :] = m * m
    loss_ref[0, 0] += jnp.sum(msq_ref[...])


def _tc_call(x_flat, embedding, W_proj):
    return pl.pallas_call(
        _tc_body,
        grid=(GRID,),
        in_specs=[
            pl.BlockSpec((BLOCK, LATENT), lambda i: (i, 0)),
            pl.BlockSpec((KCODES, LATENT), lambda i: (0, 0)),
            pl.BlockSpec((LATENT, LATENT), lambda i: (0, 0)),
        ],
        out_specs=[
            pl.BlockSpec((1, 1, BLOCK), lambda i: (i, 0, 0)),
            pl.BlockSpec((1, 1), lambda i: (0, 0),
                         memory_space=pltpu.SMEM),
            pl.BlockSpec((KCODES, LATENT), lambda i: (0, 0)),
        ],
        out_shape=[
            jax.ShapeDtypeStruct((GRID, 1, BLOCK), jnp.int32),
            jax.ShapeDtypeStruct((1, 1), jnp.float32),
            jax.ShapeDtypeStruct((KCODES, LATENT), jnp.float32),
        ],
        scratch_shapes=[
            pltpu.VMEM((1, KCODES), jnp.float32),
            pltpu.VMEM((BLOCK, 1), jnp.float32),
            pltpu.VMEM((BLOCK, 1), jnp.float32),
        ],
    )(x_flat, embedding, W_proj)


@functools.partial(
    pl.kernel,
    mesh=plsc.VectorSubcoreMesh(core_axis_name="c", subcore_axis_name="s"),
    out_type=jax.ShapeDtypeStruct((ROWS, LATENT), jnp.float32),
    scratch_types=[
        pltpu.VMEM((BPW,), jnp.int32),
        pltpu.VMEM((BPW, LATENT), jnp.float32),
        pltpu.SemaphoreType.DMA,
    ],
)
def _sc_gather(cb_hbm, idx_hbm, out_hbm, idx_v, rows_v, sem):
    wid = lax.axis_index("s") * NC + lax.axis_index("c")
    base = wid * BPW
    pltpu.sync_copy(idx_hbm.at[pl.ds(base, BPW)], idx_v)
    pltpu.async_copy(cb_hbm.at[idx_v], rows_v, sem).wait()
    pltpu.sync_copy(rows_v, out_hbm.at[pl.ds(base, BPW)])


def kernel(x, embedding, W_proj):
    x_flat = x.reshape(-1, LATENT)
    ids3, loss_acc, codebook = _tc_call(x_flat, embedding, W_proj)
    ids_flat = ids3.reshape(ROWS)
    quantized = _sc_gather(codebook, ids_flat)
    q_ste = quantized.reshape(x.shape)
    ids = ids3.reshape(x.shape[:-1])
    loss = 1.25 * loss_acc[0, 0] / (ROWS * LATENT)
    return q_ste, ids, loss
